# baseline (speedup 1.0000x reference)
import jax
import jax.numpy as jnp
from jax.experimental import pallas as pl
from jax.experimental.pallas import tpu as pltpu

_N = 10000
_F = 128
_BM = 200


def _body(x_ref, adj_ref, w_ref, o_ref):
    i = pl.program_id(0)
    adj = adj_ref[...]
    deg = jnp.sum(adj, axis=1, keepdims=True)
    acc = jnp.dot(adj, x_ref[...], preferred_element_type=jnp.float32)
    agg = acc / jnp.maximum(deg, 1e-12)
    xm = x_ref[pl.ds(i * _BM, _BM), :]
    o_ref[...] = (
        jnp.dot(xm, w_ref[:_F, :], preferred_element_type=jnp.float32)
        + jnp.dot(agg, w_ref[_F:, :], preferred_element_type=jnp.float32)
    )


def kernel(x, adj, W):
    x2 = x.reshape(_N, _F)
    adj2 = adj.reshape(_N, _N)
    out = pl.pallas_call(
        _body,
        grid=(_N // _BM,),
        in_specs=[
            pl.BlockSpec((_N, _F), lambda i: (0, 0)),
            pl.BlockSpec((_BM, _N), lambda i: (i, 0)),
            pl.BlockSpec((2 * _F, _F), lambda i: (0, 0)),
        ],
        out_specs=pl.BlockSpec((_BM, _F), lambda i: (i, 0)),
        out_shape=jax.ShapeDtypeStruct((_N, _F), jnp.float32),
        compiler_params=pltpu.CompilerParams(
            dimension_semantics=("arbitrary",),
        ),
    )(x2, adj2, W)
    return out.reshape(1, _N, _F)


# speedup vs baseline: 1.0524x; 1.0524x over previous
import jax
import jax.numpy as jnp
from jax.experimental import pallas as pl
from jax.experimental.pallas import tpu as pltpu

_N = 10000
_F = 128
_BM = 400


def _body(x_ref, adj_ref, w_ref, o_ref):
    i = pl.program_id(0)
    adj = adj_ref[...]
    deg = jnp.sum(adj, axis=1, keepdims=True)
    acc = jnp.dot(adj, x_ref[...], preferred_element_type=jnp.float32)
    agg = acc / jnp.maximum(deg, 1e-12)
    xm = x_ref[pl.ds(i * _BM, _BM), :]
    o_ref[...] = (
        jnp.dot(xm, w_ref[:_F, :], preferred_element_type=jnp.float32)
        + jnp.dot(agg, w_ref[_F:, :], preferred_element_type=jnp.float32)
    )


def kernel(x, adj, W):
    x2 = x.reshape(_N, _F)
    adj2 = adj.reshape(_N, _N)
    out = pl.pallas_call(
        _body,
        grid=(_N // _BM,),
        in_specs=[
            pl.BlockSpec((_N, _F), lambda i: (0, 0)),
            pl.BlockSpec((_BM, _N), lambda i: (i, 0)),
            pl.BlockSpec((2 * _F, _F), lambda i: (0, 0)),
        ],
        out_specs=pl.BlockSpec((_BM, _F), lambda i: (i, 0)),
        out_shape=jax.ShapeDtypeStruct((_N, _F), jnp.float32),
        compiler_params=pltpu.CompilerParams(
            dimension_semantics=("arbitrary",),
        ),
    )(x2, adj2, W)
    return out.reshape(1, _N, _F)
